# baseline (speedup 1.0000x reference)
import functools

import jax
import jax.numpy as jnp
from jax import lax
from jax.experimental import pallas as pl
from jax.experimental.pallas import tpu as pltpu
from jax.experimental.pallas import tpu_sc as plsc

_B, _N, _E, _D = 8, 2048, 65536, 128
_LANES = 16
_NSLICE = _D // _LANES
_NWORK = 32
_CHUNK = 8192
_NCH = _E // _CHUNK


def _spmm_sc(xr, edges):
    mesh = plsc.VectorSubcoreMesh(core_axis_name="c", subcore_axis_name="s")

    @functools.partial(
        pl.kernel,
        out_type=jax.ShapeDtypeStruct((_B, _NSLICE, _N, _LANES), jnp.float32),
        mesh=mesh,
        scratch_types=[
            pltpu.VMEM((_N, _LANES), jnp.float32),
            pltpu.VMEM((_N, _LANES), jnp.float32),
            pltpu.VMEM((3, _CHUNK), jnp.int32),
            pltpu.VMEM((3, _CHUNK), jnp.int32),
            pltpu.SemaphoreType.DMA,
            pltpu.SemaphoreType.DMA,
            pltpu.SemaphoreType.DMA,
        ],
        compiler_params=pltpu.CompilerParams(
            use_tc_tiling_on_sc=False, needs_layout_passes=False),
    )
    def spmm_kernel(xr_hbm, e_hbm, a_hbm, xv, acc, eb0, eb1, sem0, sem1, semx):
        w = lax.axis_index("s") * 2 + lax.axis_index("c")

        _XS, _ES = 8, 2
        _XQ, _EQ = _N // _XS, _CHUNK // _ES

        def x_copies(b, ds):
            return [pltpu.make_async_copy(
                        xr_hbm.at[b, ds, pl.ds(q * _XQ, _XQ)],
                        xv.at[pl.ds(q * _XQ, _XQ)], semx)
                    for q in range(_XS)]

        def e_copies(eb, sem, b, ck):
            return [pltpu.make_async_copy(
                        e_hbm.at[b, ck, j, pl.ds(h * _EQ, _EQ)],
                        eb.at[j, pl.ds(h * _EQ, _EQ)], sem)
                    for j in range(3) for h in range(_ES)]

        def process(eb):
            @plsc.parallel_loop(0, _CHUNK, step=_LANES, unroll=4)
            def _group(g):
                sv = eb[0, pl.ds(g, _LANES)]
                dv = eb[1, pl.ds(g, _LANES)]
                vv = plsc.bitcast(eb[2, pl.ds(g, _LANES)], jnp.float32)
                for r in range(_LANES):
                    row = jnp.full((_LANES,), r, jnp.int32)
                    xg = plsc.load_gather(xv, [sv, row])
                    plsc.addupdate_scatter(acc, [dv, row], xg * vv)

        @pl.loop(0, 2)
        def _item(k):
            itm = w + _NWORK * k
            b = itm // _NSLICE
            ds = itm % _NSLICE

            for c in x_copies(b, ds):
                c.start()
            for c in e_copies(eb0, sem0, b, 0):
                c.start()

            @pl.loop(0, _N)
            def _zero(i):
                acc[i, :] = jnp.zeros((_LANES,), jnp.float32)

            for c in x_copies(b, ds):
                c.wait()

            @pl.loop(0, _NCH // 2)
            def _pair(i):
                ck = 2 * i
                for c in e_copies(eb1, sem1, b, ck + 1):
                    c.start()
                for c in e_copies(eb0, sem0, b, ck):
                    c.wait()
                process(eb0)

                @pl.when(ck + 2 < _NCH)
                def _pref():
                    for c in e_copies(eb0, sem0, b, ck + 2):
                        c.start()

                for c in e_copies(eb1, sem1, b, ck + 1):
                    c.wait()
                process(eb1)

            pltpu.sync_copy(acc, a_hbm.at[b, ds])

    return spmm_kernel(xr, edges)


_BN = 256


def _gru_body(a_ref, x_ref, m_ref, wz0, wz1, wr0, wr1, wh0, wh1,
              bz, br, bh, o_ref):
    a = a_ref[0]
    xb = x_ref[0]
    m = m_ref[0]
    dot = functools.partial(jnp.dot, preferred_element_type=jnp.float32)
    z = jax.nn.sigmoid(dot(a, wz0[...]) + dot(xb, wz1[...]) + bz[...])
    r = jax.nn.sigmoid(dot(a, wr0[...]) + dot(xb, wr1[...]) + br[...])
    h = jnp.tanh(m * (dot(a, wh0[...]) + dot(r * xb, wh1[...]) + bh[...]))
    o_ref[0] = z * h + (1.0 - z) * xb


def _gru_tc(a, x, mask, wz0, wz1, wr0, wr1, wh0, wh1, bz, br, bh):
    wspec = pl.BlockSpec((_D, _D), lambda b, i: (0, 0))
    bspec = pl.BlockSpec((1, _D), lambda b, i: (0, 0))
    blk = pl.BlockSpec((1, _BN, _D), lambda b, i: (b, i, 0))
    mblk = pl.BlockSpec((1, _BN, 1), lambda b, i: (b, i, 0))
    return pl.pallas_call(
        _gru_body,
        grid=(_B, _N // _BN),
        in_specs=[blk, blk, mblk] + [wspec] * 6 + [bspec] * 3,
        out_specs=blk,
        out_shape=jax.ShapeDtypeStruct((_B, _N, _D), jnp.float32),
    )(a, x, mask, wz0, wz1, wr0, wr1, wh0, wh1, bz, br, bh)


def kernel(adj_indices, adj_values, x, mask,
           z0_weight, z0_bias, z1_weight, z1_bias,
           r0_weight, r0_bias, r1_weight, r1_bias,
           h0_weight, h0_bias, h1_weight, h1_bias):
    dst = adj_indices[:, 0, :].astype(jnp.int32)
    src = adj_indices[:, 1, :].astype(jnp.int32)
    val_bits = lax.bitcast_convert_type(adj_values, jnp.int32)
    edges = jnp.stack([a.reshape(_B, _NCH, _CHUNK) for a in (src, dst, val_bits)],
                      axis=2)
    xr = x.reshape(_B, _N, _NSLICE, _LANES).transpose(0, 2, 1, 3)
    aT = _spmm_sc(xr, edges)
    a = aT.transpose(0, 2, 1, 3).reshape(_B, _N, _D)
    bz = (z0_bias + z1_bias).reshape(1, _D)
    br = (r0_bias + r1_bias).reshape(1, _D)
    bh = (h0_bias + h1_bias).reshape(1, _D)
    return _gru_tc(a, x, mask, z0_weight, z1_weight, r0_weight, r1_weight,
                   h0_weight, h1_weight, bz, br, bh)


# speedup vs baseline: 2.8872x; 2.8872x over previous
import functools

import jax
import jax.numpy as jnp
from jax import lax
from jax.experimental import pallas as pl
from jax.experimental.pallas import tpu as pltpu
from jax.experimental.pallas import tpu_sc as plsc

_B, _N, _E, _D = 8, 2048, 65536, 128
_LANES = 16
_NSLICE = _D // _LANES
_NWORK = 32
_CHUNK = 8192
_NCH = _E // _CHUNK


def _spmm_sc(xr, edges):
    mesh = plsc.VectorSubcoreMesh(core_axis_name="c", subcore_axis_name="s")

    @functools.partial(
        pl.kernel,
        out_type=jax.ShapeDtypeStruct((_B, _NSLICE, _LANES, _N), jnp.float32),
        mesh=mesh,
        scratch_types=[
            pltpu.VMEM((_LANES, _N), jnp.float32),
            pltpu.VMEM((_LANES, _N), jnp.float32),
            pltpu.VMEM((3, _CHUNK), jnp.int32),
            pltpu.VMEM((3, _CHUNK), jnp.int32),
            pltpu.SemaphoreType.DMA,
            pltpu.SemaphoreType.DMA,
            pltpu.SemaphoreType.DMA,
        ],
        compiler_params=pltpu.CompilerParams(
            use_tc_tiling_on_sc=False, needs_layout_passes=False),
    )
    def spmm_kernel(xr_hbm, e_hbm, a_hbm, xv, acc, eb0, eb1, sem0, sem1, semx):
        w = lax.axis_index("s") * 2 + lax.axis_index("c")

        _XS, _ES = 8, 2
        _XQ, _EQ = _LANES // _XS, _CHUNK // _ES

        def x_copies(b, ds):
            return [pltpu.make_async_copy(
                        xr_hbm.at[b, ds, pl.ds(q * _XQ, _XQ)],
                        xv.at[pl.ds(q * _XQ, _XQ)], semx)
                    for q in range(_XS)]

        def e_copies(eb, sem, b, ck):
            return [pltpu.make_async_copy(
                        e_hbm.at[b, ck, j, pl.ds(h * _EQ, _EQ)],
                        eb.at[j, pl.ds(h * _EQ, _EQ)], sem)
                    for j in range(3) for h in range(_ES)]

        def process(eb):
            @plsc.parallel_loop(0, _CHUNK, step=_LANES, unroll=4)
            def _group(g):
                sv = eb[0, pl.ds(g, _LANES)]
                dv = eb[1, pl.ds(g, _LANES)]
                vv = plsc.bitcast(eb[2, pl.ds(g, _LANES)], jnp.float32)
                for r in range(_LANES):
                    row = jnp.full((_LANES,), r, jnp.int32)
                    xg = plsc.load_gather(xv, [row, sv])
                    plsc.addupdate_scatter(acc, [row, dv], xg * vv)

        @pl.loop(0, 2)
        def _item(k):
            itm = w + _NWORK * k
            b = itm // _NSLICE
            ds = itm % _NSLICE

            for c in x_copies(b, ds):
                c.start()
            for c in e_copies(eb0, sem0, b, 0):
                c.start()

            @pl.loop(0, _N, step=_LANES)
            def _zero(i):
                for r in range(_LANES):
                    acc[r, pl.ds(i, _LANES)] = jnp.zeros((_LANES,), jnp.float32)

            for c in x_copies(b, ds):
                c.wait()

            @pl.loop(0, _NCH // 2)
            def _pair(i):
                ck = 2 * i
                for c in e_copies(eb1, sem1, b, ck + 1):
                    c.start()
                for c in e_copies(eb0, sem0, b, ck):
                    c.wait()
                process(eb0)

                @pl.when(ck + 2 < _NCH)
                def _pref():
                    for c in e_copies(eb0, sem0, b, ck + 2):
                        c.start()

                for c in e_copies(eb1, sem1, b, ck + 1):
                    c.wait()
                process(eb1)

            pltpu.sync_copy(acc, a_hbm.at[b, ds])

    return spmm_kernel(xr, edges)


_BN = 256


def _gru_body(a_ref, x_ref, m_ref, wz0, wz1, wr0, wr1, wh0, wh1,
              bz, br, bh, o_ref):
    a = a_ref[0]
    xb = x_ref[0]
    m = m_ref[0]
    dot = functools.partial(jnp.dot, preferred_element_type=jnp.float32)
    z = jax.nn.sigmoid(dot(a, wz0[...]) + dot(xb, wz1[...]) + bz[...])
    r = jax.nn.sigmoid(dot(a, wr0[...]) + dot(xb, wr1[...]) + br[...])
    h = jnp.tanh(m * (dot(a, wh0[...]) + dot(r * xb, wh1[...]) + bh[...]))
    o_ref[0] = z * h + (1.0 - z) * xb


def _gru_tc(a, x, mask, wz0, wz1, wr0, wr1, wh0, wh1, bz, br, bh):
    wspec = pl.BlockSpec((_D, _D), lambda b, i: (0, 0))
    bspec = pl.BlockSpec((1, _D), lambda b, i: (0, 0))
    blk = pl.BlockSpec((1, _BN, _D), lambda b, i: (b, i, 0))
    mblk = pl.BlockSpec((1, _BN, 1), lambda b, i: (b, i, 0))
    return pl.pallas_call(
        _gru_body,
        grid=(_B, _N // _BN),
        in_specs=[blk, blk, mblk] + [wspec] * 6 + [bspec] * 3,
        out_specs=blk,
        out_shape=jax.ShapeDtypeStruct((_B, _N, _D), jnp.float32),
    )(a, x, mask, wz0, wz1, wr0, wr1, wh0, wh1, bz, br, bh)


def kernel(adj_indices, adj_values, x, mask,
           z0_weight, z0_bias, z1_weight, z1_bias,
           r0_weight, r0_bias, r1_weight, r1_bias,
           h0_weight, h0_bias, h1_weight, h1_bias):
    dst = adj_indices[:, 0, :].astype(jnp.int32)
    src = adj_indices[:, 1, :].astype(jnp.int32)
    val_bits = lax.bitcast_convert_type(adj_values, jnp.int32)
    edges = jnp.stack([a.reshape(_B, _NCH, _CHUNK) for a in (src, dst, val_bits)],
                      axis=2)
    xr = x.reshape(_B, _N, _NSLICE, _LANES).transpose(0, 2, 3, 1)
    aT = _spmm_sc(xr, edges)
    a = aT.transpose(0, 3, 1, 2).reshape(_B, _N, _D)
    bz = (z0_bias + z1_bias).reshape(1, _D)
    br = (r0_bias + r1_bias).reshape(1, _D)
    bh = (h0_bias + h1_bias).reshape(1, _D)
    return _gru_tc(a, x, mask, z0_weight, z1_weight, r0_weight, r1_weight,
                   h0_weight, h1_weight, bz, br, bh)
